# baseline (speedup 1.0000x reference)
import jax
import jax.numpy as jnp
from jax.experimental import pallas as pl
from jax.experimental.pallas import tpu as pltpu

_N = 4096
_D = 64
_NP = 4
_NA = 32
_H = 64
_TOPK = 32
_ZD = _NP * _D + _H
_BLOCK = 256
_CHUNK = 128


def _z_kernel(ctx_ref, pe_ref, w_ref, wpe_ref, ps_ref, z_ref):
    ctx = ctx_ref[...]
    w = w_ref[...]
    for p in range(_NP):
        x = ctx * w[p, :][None, :]
        nrm = jnp.sqrt(jnp.sum(x * x, axis=1, keepdims=True))
        x = x / jnp.maximum(nrm, 1e-12)
        z_ref[:, p * _D:(p + 1) * _D] = x
    pe = jax.lax.dot_general(
        pe_ref[...], wpe_ref[...], (((1,), (0,)), ((), ())),
        preferred_element_type=jnp.float32)
    z_ref[:, _NP * _D:] = pe * ps_ref[0, 0]


def _to_key(v):
    b = jax.lax.bitcast_convert_type(v, jnp.int32)
    return b ^ jax.lax.shift_right_arithmetic(b, 31).__and__(0x7FFFFFFF)


def _to_float(k):
    b = k ^ jax.lax.shift_right_arithmetic(k, 31).__and__(0x7FFFFFFF)
    return jax.lax.bitcast_convert_type(b, jnp.float32)


def _topk_kernel(zrow_ref, zall_ref, gpr_ref, os_ref, out_ref, v_ref):
    zr = zrow_ref[...]
    za = zall_ref[...]
    dn = (((1,), (1,)), ((), ()))
    st = jax.lax.dot_general(
        za, zr, dn, preferred_element_type=jnp.float32)
    st = st * os_ref[0, 0]
    st = st * gpr_ref[...]
    v_ref[...] = st

    m = jnp.max(st.reshape(_N // _CHUNK, _CHUNK, _BLOCK), axis=1)
    mk = _to_key(m)
    lo = jnp.min(mk, axis=0, keepdims=True)
    hi = jnp.max(mk, axis=0, keepdims=True) + 1

    def cond(state):
        lo, hi = state
        return jnp.any(jax.lax.shift_right_logical(hi - lo, 1) != 0)

    def body(state):
        lo, hi = state
        w = hi - lo
        half = jax.lax.shift_right_logical(w, 1)
        open_ = half != 0
        d = jax.lax.shift_right_logical(w, 2)
        deg = d == 0
        t1 = lo + jnp.where(deg, half, d)
        t2 = lo + jnp.where(deg, half, 2 * d)
        t3 = lo + jnp.where(deg, half, 3 * d)
        f1 = _to_float(t1)
        f2 = _to_float(t2)
        f3 = _to_float(t3)
        v = v_ref[...]
        b1 = (v >= f1).astype(jnp.int32)
        b2 = (v >= f2).astype(jnp.int32)
        b3 = (v >= f3).astype(jnp.int32)
        c1 = jnp.sum(b1, axis=0, keepdims=True)
        c2 = jnp.sum(b2, axis=0, keepdims=True)
        c3 = jnp.sum(b3, axis=0, keepdims=True)
        g1 = c1 >= _TOPK
        g2 = c2 >= _TOPK
        g3 = c3 >= _TOPK
        nlo = jnp.where(g3, t3, jnp.where(g2, t2, jnp.where(g1, t1, lo)))
        nhi = jnp.where(g3, hi, jnp.where(g2, t3, jnp.where(g1, t2, t1)))
        e1 = c1 == _TOPK
        e2 = c2 == _TOPK
        e3 = c3 == _TOPK
        anyeq = jnp.logical_or(e1, jnp.logical_or(e2, e3))
        clo = jnp.where(e3, t3, jnp.where(e2, t2, t1))
        lo = jnp.where(open_, jnp.where(anyeq, clo, nlo), lo)
        hi = jnp.where(open_, jnp.where(anyeq, clo + 1, nhi), hi)
        return lo, hi

    lo, hi = jax.lax.while_loop(cond, body, (lo, hi))
    t = _to_float(lo)
    v = v_ref[...]
    out_ref[...] = jnp.where(v >= t, v, 0.0).T


def kernel(context, position_encoding, gpr_rank, position_flag, W, Wpe):
    flag = jnp.asarray(position_flag)
    ps = jnp.where(flag == 1, 2.0, 0.0).astype(jnp.float32).reshape(1, 1)
    os_ = jnp.where(flag == 1, 0.125, 0.25).astype(jnp.float32).reshape(1, 1)
    gpr_row = gpr_rank.reshape(1, _N)

    z = pl.pallas_call(
        _z_kernel,
        out_shape=jax.ShapeDtypeStruct((_N, _ZD), jnp.float32),
    )(context, position_encoding, W, Wpe, ps)

    out = pl.pallas_call(
        _topk_kernel,
        grid=(_N // _BLOCK,),
        in_specs=[
            pl.BlockSpec((_BLOCK, _ZD), lambda i: (i, 0)),
            pl.BlockSpec((_N, _ZD), lambda i: (0, 0)),
            pl.BlockSpec((1, _BLOCK), lambda i: (0, i)),
            pl.BlockSpec((1, 1), lambda i: (0, 0)),
        ],
        out_specs=pl.BlockSpec((_BLOCK, _N), lambda i: (i, 0)),
        out_shape=jax.ShapeDtypeStruct((_N, _N), jnp.float32),
        scratch_shapes=[pltpu.VMEM((_N, _BLOCK), jnp.float32)],
        compiler_params=pltpu.CompilerParams(
            dimension_semantics=("arbitrary",)),
    )(z, z, gpr_row, os_)
    return out


# speedup vs baseline: 1.3000x; 1.3000x over previous
import jax
import jax.numpy as jnp
from jax.experimental import pallas as pl
from jax.experimental.pallas import tpu as pltpu

_N = 4096
_D = 64
_NP = 4
_NA = 32
_H = 64
_TOPK = 32
_ZD = _NP * _D + _H
_BLOCK = 512
_CHUNK = 128


def _z_kernel(ctx_ref, pe_ref, w_ref, wpe_ref, ps_ref, z_ref):
    ctx = ctx_ref[...]
    w = w_ref[...]
    for p in range(_NP):
        x = ctx * w[p, :][None, :]
        nrm = jnp.sqrt(jnp.sum(x * x, axis=1, keepdims=True))
        x = x / jnp.maximum(nrm, 1e-12)
        z_ref[:, p * _D:(p + 1) * _D] = x
    pe = jax.lax.dot_general(
        pe_ref[...], wpe_ref[...], (((1,), (0,)), ((), ())),
        preferred_element_type=jnp.float32)
    z_ref[:, _NP * _D:] = pe * ps_ref[0, 0]


def _to_key(v):
    b = jax.lax.bitcast_convert_type(v, jnp.int32)
    return b ^ jax.lax.shift_right_arithmetic(b, 31).__and__(0x7FFFFFFF)


def _to_float(k):
    b = k ^ jax.lax.shift_right_arithmetic(k, 31).__and__(0x7FFFFFFF)
    return jax.lax.bitcast_convert_type(b, jnp.float32)


def _topk_kernel(zrow_ref, zall_ref, gpr_ref, os_ref, out_ref, v_ref):
    zr = zrow_ref[...]
    za = zall_ref[...]
    dn = (((1,), (1,)), ((), ()))
    st = jax.lax.dot_general(
        za, zr, dn, preferred_element_type=jnp.float32)
    st = st * os_ref[0, 0]
    st = st * gpr_ref[...]
    v_ref[...] = st

    m = jnp.max(st.reshape(_N // _CHUNK, _CHUNK, _BLOCK), axis=1)
    mk = _to_key(m)
    lo = jnp.min(mk, axis=0, keepdims=True)
    hi = jnp.max(mk, axis=0, keepdims=True) + 1

    def cond(state):
        lo, hi = state
        return jnp.any(jax.lax.shift_right_logical(hi - lo, 1) != 0)

    def body(state):
        lo, hi = state
        half = jax.lax.shift_right_logical(hi - lo, 1)
        open_ = half != 0
        mid = lo + half
        tmid = _to_float(mid)
        v = v_ref[...]
        c = jnp.sum((v >= tmid).astype(jnp.int32), axis=0, keepdims=True)
        ge = c >= _TOPK
        eq = c == _TOPK
        lo = jnp.where(jnp.logical_and(open_, ge), mid, lo)
        hi = jnp.where(
            jnp.logical_and(open_, jnp.logical_not(ge)), mid,
            jnp.where(jnp.logical_and(open_, eq), mid + 1, hi))
        return lo, hi

    lo, hi = jax.lax.while_loop(cond, body, (lo, hi))
    t = _to_float(lo)
    v = v_ref[...]
    out_ref[...] = jnp.where(v >= t, v, 0.0).T


def kernel(context, position_encoding, gpr_rank, position_flag, W, Wpe):
    flag = jnp.asarray(position_flag)
    ps = jnp.where(flag == 1, 2.0, 0.0).astype(jnp.float32).reshape(1, 1)
    os_ = jnp.where(flag == 1, 0.125, 0.25).astype(jnp.float32).reshape(1, 1)
    gpr_row = gpr_rank.reshape(1, _N)

    z = pl.pallas_call(
        _z_kernel,
        out_shape=jax.ShapeDtypeStruct((_N, _ZD), jnp.float32),
    )(context, position_encoding, W, Wpe, ps)

    out = pl.pallas_call(
        _topk_kernel,
        grid=(_N // _BLOCK,),
        in_specs=[
            pl.BlockSpec((_BLOCK, _ZD), lambda i: (i, 0)),
            pl.BlockSpec((_N, _ZD), lambda i: (0, 0)),
            pl.BlockSpec((1, _BLOCK), lambda i: (0, i)),
            pl.BlockSpec((1, 1), lambda i: (0, 0)),
        ],
        out_specs=pl.BlockSpec((_BLOCK, _N), lambda i: (i, 0)),
        out_shape=jax.ShapeDtypeStruct((_N, _N), jnp.float32),
        scratch_shapes=[pltpu.VMEM((_N, _BLOCK), jnp.float32)],
        compiler_params=pltpu.CompilerParams(
            dimension_semantics=("arbitrary",)),
    )(z, z, gpr_row, os_)
    return out


# speedup vs baseline: 1.3095x; 1.0074x over previous
import jax
import jax.numpy as jnp
from jax.experimental import pallas as pl
from jax.experimental.pallas import tpu as pltpu

_N = 4096
_D = 64
_NP = 4
_NA = 32
_H = 64
_TOPK = 32
_ZD = _NP * _D + _H
_BLOCK = 1024
_CHUNK = 128


def _z_kernel(ctx_ref, pe_ref, w_ref, wpe_ref, ps_ref, z_ref):
    ctx = ctx_ref[...]
    w = w_ref[...]
    for p in range(_NP):
        x = ctx * w[p, :][None, :]
        nrm = jnp.sqrt(jnp.sum(x * x, axis=1, keepdims=True))
        x = x / jnp.maximum(nrm, 1e-12)
        z_ref[:, p * _D:(p + 1) * _D] = x
    pe = jax.lax.dot_general(
        pe_ref[...], wpe_ref[...], (((1,), (0,)), ((), ())),
        preferred_element_type=jnp.float32)
    z_ref[:, _NP * _D:] = pe * ps_ref[0, 0]


def _to_key(v):
    b = jax.lax.bitcast_convert_type(v, jnp.int32)
    return b ^ jax.lax.shift_right_arithmetic(b, 31).__and__(0x7FFFFFFF)


def _to_float(k):
    b = k ^ jax.lax.shift_right_arithmetic(k, 31).__and__(0x7FFFFFFF)
    return jax.lax.bitcast_convert_type(b, jnp.float32)


def _topk_kernel(zrow_ref, zall_ref, gpr_ref, os_ref, out_ref, v_ref):
    zr = zrow_ref[...]
    za = zall_ref[...]
    dn = (((1,), (1,)), ((), ()))
    st = jax.lax.dot_general(
        za, zr, dn, preferred_element_type=jnp.float32)
    st = st * os_ref[0, 0]
    st = st * gpr_ref[...]
    v_ref[...] = st

    m = jnp.max(st.reshape(_N // _CHUNK, _CHUNK, _BLOCK), axis=1)
    mk = _to_key(m)
    lo = jnp.min(mk, axis=0, keepdims=True)
    hi = jnp.max(mk, axis=0, keepdims=True) + 1

    def cond(state):
        lo, hi = state
        return jnp.any(jax.lax.shift_right_logical(hi - lo, 1) != 0)

    def body(state):
        lo, hi = state
        half = jax.lax.shift_right_logical(hi - lo, 1)
        open_ = half != 0
        mid = lo + half
        tmid = _to_float(mid)
        v = v_ref[...]
        c = jnp.sum((v >= tmid).astype(jnp.int32), axis=0, keepdims=True)
        ge = c >= _TOPK
        eq = c == _TOPK
        lo = jnp.where(jnp.logical_and(open_, ge), mid, lo)
        hi = jnp.where(
            jnp.logical_and(open_, jnp.logical_not(ge)), mid,
            jnp.where(jnp.logical_and(open_, eq), mid + 1, hi))
        return lo, hi

    lo, hi = jax.lax.while_loop(cond, body, (lo, hi))
    t = _to_float(lo)
    v = v_ref[...]
    out_ref[...] = jnp.where(v >= t, v, 0.0).T


def kernel(context, position_encoding, gpr_rank, position_flag, W, Wpe):
    flag = jnp.asarray(position_flag)
    ps = jnp.where(flag == 1, 2.0, 0.0).astype(jnp.float32).reshape(1, 1)
    os_ = jnp.where(flag == 1, 0.125, 0.25).astype(jnp.float32).reshape(1, 1)
    gpr_row = gpr_rank.reshape(1, _N)

    z = pl.pallas_call(
        _z_kernel,
        out_shape=jax.ShapeDtypeStruct((_N, _ZD), jnp.float32),
    )(context, position_encoding, W, Wpe, ps)

    out = pl.pallas_call(
        _topk_kernel,
        grid=(_N // _BLOCK,),
        in_specs=[
            pl.BlockSpec((_BLOCK, _ZD), lambda i: (i, 0)),
            pl.BlockSpec((_N, _ZD), lambda i: (0, 0)),
            pl.BlockSpec((1, _BLOCK), lambda i: (0, i)),
            pl.BlockSpec((1, 1), lambda i: (0, 0)),
        ],
        out_specs=pl.BlockSpec((_BLOCK, _N), lambda i: (i, 0)),
        out_shape=jax.ShapeDtypeStruct((_N, _N), jnp.float32),
        scratch_shapes=[pltpu.VMEM((_N, _BLOCK), jnp.float32)],
        compiler_params=pltpu.CompilerParams(
            dimension_semantics=("arbitrary",),
            vmem_limit_bytes=100 * 1024 * 1024),
    )(z, z, gpr_row, os_)
    return out


# speedup vs baseline: 1.4752x; 1.1265x over previous
import jax
import jax.numpy as jnp
from jax.experimental import pallas as pl
from jax.experimental.pallas import tpu as pltpu

_N = 4096
_D = 64
_NP = 4
_NA = 32
_H = 64
_TOPK = 32
_ZD = _NP * _D + _H
_BLOCK = 512
_CHUNK = 128


def _z_kernel(ctx_ref, pe_ref, w_ref, wpe_ref, ps_ref, z_ref):
    ctx = ctx_ref[...]
    w = w_ref[...]
    for p in range(_NP):
        x = ctx * w[p, :][None, :]
        nrm = jnp.sqrt(jnp.sum(x * x, axis=1, keepdims=True))
        x = x / jnp.maximum(nrm, 1e-12)
        z_ref[:, p * _D:(p + 1) * _D] = x
    pe = jax.lax.dot_general(
        pe_ref[...], wpe_ref[...], (((1,), (0,)), ((), ())),
        preferred_element_type=jnp.float32)
    z_ref[:, _NP * _D:] = pe * ps_ref[0, 0]


def _to_key(v):
    b = jax.lax.bitcast_convert_type(v, jnp.int32)
    return b ^ jax.lax.shift_right_arithmetic(b, 31).__and__(0x7FFFFFFF)


def _to_float(k):
    b = k ^ jax.lax.shift_right_arithmetic(k, 31).__and__(0x7FFFFFFF)
    return jax.lax.bitcast_convert_type(b, jnp.float32)


def _topk_kernel(zrow_ref, zall_ref, gpr_ref, os_ref, out_ref, v_ref):
    zr = zrow_ref[...]
    za = zall_ref[...]
    dn = (((1,), (1,)), ((), ()))
    st = jax.lax.dot_general(
        za, zr, dn, preferred_element_type=jnp.float32)
    st = st * os_ref[0, 0]
    st = st * gpr_ref[...]
    v_ref[...] = st

    v3 = st.reshape(_N // _CHUNK, _CHUNK, _BLOCK)
    ninf = jnp.float32(-jnp.inf)
    c1 = jnp.max(v3, axis=1)
    e1 = v3 == c1[:, None, :]
    c2 = jnp.max(jnp.where(e1, ninf, v3), axis=1)
    e2 = jnp.logical_or(e1, v3 == c2[:, None, :])
    c3 = jnp.max(jnp.where(e2, ninf, v3), axis=1)
    e3 = jnp.logical_or(e2, v3 == c3[:, None, :])
    c4 = jnp.max(jnp.where(e3, ninf, v3), axis=1)
    cache = jnp.concatenate([c1, c2, c3, c4], axis=0)

    ck = _to_key(cache)
    lo = jnp.min(ck, axis=0, keepdims=True)
    hi = jnp.max(ck, axis=0, keepdims=True) + 1

    def ccond(state):
        lo, hi = state
        return jnp.any(jax.lax.shift_right_logical(hi - lo, 1) != 0)

    def cbody(state):
        lo, hi = state
        half = jax.lax.shift_right_logical(hi - lo, 1)
        open_ = half != 0
        mid = lo + half
        c = jnp.sum((ck >= mid).astype(jnp.int32), axis=0, keepdims=True)
        ge = c >= _TOPK
        eq = c == _TOPK
        lo = jnp.where(jnp.logical_and(open_, ge), mid, lo)
        hi = jnp.where(
            jnp.logical_and(open_, jnp.logical_not(ge)), mid,
            jnp.where(jnp.logical_and(open_, eq), mid + 1, hi))
        return lo, hi

    lo, hi = jax.lax.while_loop(ccond, cbody, (lo, hi))
    fkey0 = lo - 1

    pinf = jnp.float32(jnp.inf)

    def rcond(state):
        fkey, tkey, open_ = state
        return jnp.any(open_ != 0)

    def rbody(state):
        fkey, tkey, open_ = state
        f = _to_float(fkey)
        v = v_ref[...]
        mask = v > f
        vm = jnp.where(mask, v, pinf)
        c = jnp.sum(mask.astype(jnp.int32), axis=0, keepdims=True)
        mn = jnp.min(vm, axis=0, keepdims=True)
        mnk = _to_key(mn)
        gt = c > _TOPK
        eq = c == _TOPK
        lt = c < _TOPK
        opn = open_ != 0
        tkey = jnp.where(jnp.logical_and(opn, eq), mnk,
                         jnp.where(jnp.logical_and(opn, lt), fkey, tkey))
        fkey = jnp.where(jnp.logical_and(opn, gt), mnk, fkey)
        open_ = jnp.where(jnp.logical_and(opn, gt), open_, 0)
        return fkey, tkey, open_

    ones = jnp.ones_like(fkey0)
    fkey, tkey, _ = jax.lax.while_loop(
        rcond, rbody, (fkey0, fkey0, ones))
    t = _to_float(tkey)
    v = v_ref[...]
    out_ref[...] = jnp.where(v >= t, v, 0.0).T


def kernel(context, position_encoding, gpr_rank, position_flag, W, Wpe):
    flag = jnp.asarray(position_flag)
    ps = jnp.where(flag == 1, 2.0, 0.0).astype(jnp.float32).reshape(1, 1)
    os_ = jnp.where(flag == 1, 0.125, 0.25).astype(jnp.float32).reshape(1, 1)
    gpr_row = gpr_rank.reshape(1, _N)

    z = pl.pallas_call(
        _z_kernel,
        out_shape=jax.ShapeDtypeStruct((_N, _ZD), jnp.float32),
    )(context, position_encoding, W, Wpe, ps)

    out = pl.pallas_call(
        _topk_kernel,
        grid=(_N // _BLOCK,),
        in_specs=[
            pl.BlockSpec((_BLOCK, _ZD), lambda i: (i, 0)),
            pl.BlockSpec((_N, _ZD), lambda i: (0, 0)),
            pl.BlockSpec((1, _BLOCK), lambda i: (0, i)),
            pl.BlockSpec((1, 1), lambda i: (0, 0)),
        ],
        out_specs=pl.BlockSpec((_BLOCK, _N), lambda i: (i, 0)),
        out_shape=jax.ShapeDtypeStruct((_N, _N), jnp.float32),
        scratch_shapes=[pltpu.VMEM((_N, _BLOCK), jnp.float32)],
        compiler_params=pltpu.CompilerParams(
            dimension_semantics=("arbitrary",),
            vmem_limit_bytes=100 * 1024 * 1024),
    )(z, z, gpr_row, os_)
    return out


# speedup vs baseline: 1.5873x; 1.0760x over previous
import jax
import jax.numpy as jnp
from jax.experimental import pallas as pl
from jax.experimental.pallas import tpu as pltpu

_N = 4096
_D = 64
_NP = 4
_NA = 32
_H = 64
_TOPK = 32
_ZD = _NP * _D + _H
_BLOCK = 512
_CHUNK = 64


def _z_kernel(ctx_ref, pe_ref, w_ref, wpe_ref, ps_ref, z_ref):
    ctx = ctx_ref[...]
    w = w_ref[...]
    for p in range(_NP):
        x = ctx * w[p, :][None, :]
        nrm = jnp.sqrt(jnp.sum(x * x, axis=1, keepdims=True))
        x = x / jnp.maximum(nrm, 1e-12)
        z_ref[:, p * _D:(p + 1) * _D] = x
    pe = jax.lax.dot_general(
        pe_ref[...], wpe_ref[...], (((1,), (0,)), ((), ())),
        preferred_element_type=jnp.float32)
    z_ref[:, _NP * _D:] = pe * ps_ref[0, 0]


def _to_key(v):
    b = jax.lax.bitcast_convert_type(v, jnp.int32)
    return b ^ jax.lax.shift_right_arithmetic(b, 31).__and__(0x7FFFFFFF)


def _to_float(k):
    b = k ^ jax.lax.shift_right_arithmetic(k, 31).__and__(0x7FFFFFFF)
    return jax.lax.bitcast_convert_type(b, jnp.float32)


def _topk_kernel(zrow_ref, zall_ref, gpr_ref, os_ref, out_ref, v_ref):
    zr = zrow_ref[...]
    za = zall_ref[...]
    dn = (((1,), (1,)), ((), ()))
    st = jax.lax.dot_general(
        za, zr, dn, preferred_element_type=jnp.float32)
    st = st * os_ref[0, 0]
    st = st * gpr_ref[...]
    v_ref[...] = st

    v3 = st.reshape(_N // _CHUNK, _CHUNK, _BLOCK)
    ninf = jnp.float32(-jnp.inf)
    c1 = jnp.max(v3, axis=1)
    e1 = v3 == c1[:, None, :]
    c2 = jnp.max(jnp.where(e1, ninf, v3), axis=1)
    e2 = jnp.logical_or(e1, v3 == c2[:, None, :])
    c3 = jnp.max(jnp.where(e2, ninf, v3), axis=1)
    cache = jnp.concatenate([c1, c2, c3], axis=0)

    ck = _to_key(cache)
    lo = jnp.min(ck, axis=0, keepdims=True)
    hi = jnp.max(ck, axis=0, keepdims=True) + 1

    def ccond(state):
        lo, hi = state
        return jnp.any(jax.lax.shift_right_logical(hi - lo, 1) != 0)

    def cbody(state):
        lo, hi = state
        half = jax.lax.shift_right_logical(hi - lo, 1)
        open_ = half != 0
        mid = lo + half
        c = jnp.sum((ck >= mid).astype(jnp.int32), axis=0, keepdims=True)
        ge = c >= _TOPK
        eq = c == _TOPK
        lo = jnp.where(jnp.logical_and(open_, ge), mid, lo)
        hi = jnp.where(
            jnp.logical_and(open_, jnp.logical_not(ge)), mid,
            jnp.where(jnp.logical_and(open_, eq), mid + 1, hi))
        return lo, hi

    lo, hi = jax.lax.while_loop(ccond, cbody, (lo, hi))
    fkey0 = lo - 1

    pinf = jnp.float32(jnp.inf)

    def rcond(state):
        fkey, tkey, open_ = state
        return jnp.any(open_ != 0)

    def rbody(state):
        fkey, tkey, open_ = state
        f = _to_float(fkey)
        v = v_ref[...]
        mask = v > f
        vm = jnp.where(mask, v, pinf)
        c = jnp.sum(mask.astype(jnp.int32), axis=0, keepdims=True)
        mn = jnp.min(vm, axis=0, keepdims=True)
        mnk = _to_key(mn)
        gt = c > _TOPK
        eq = c == _TOPK
        lt = c < _TOPK
        opn = open_ != 0
        tkey = jnp.where(jnp.logical_and(opn, eq), mnk,
                         jnp.where(jnp.logical_and(opn, lt), fkey, tkey))
        fkey = jnp.where(jnp.logical_and(opn, gt), mnk, fkey)
        open_ = jnp.where(jnp.logical_and(opn, gt), open_, 0)
        return fkey, tkey, open_

    ones = jnp.ones_like(fkey0)
    fkey, tkey, _ = jax.lax.while_loop(
        rcond, rbody, (fkey0, fkey0, ones))
    t = _to_float(tkey)
    v = v_ref[...]
    out_ref[...] = jnp.where(v >= t, v, 0.0).T


def kernel(context, position_encoding, gpr_rank, position_flag, W, Wpe):
    flag = jnp.asarray(position_flag)
    ps = jnp.where(flag == 1, 2.0, 0.0).astype(jnp.float32).reshape(1, 1)
    os_ = jnp.where(flag == 1, 0.125, 0.25).astype(jnp.float32).reshape(1, 1)
    gpr_row = gpr_rank.reshape(1, _N)

    z = pl.pallas_call(
        _z_kernel,
        out_shape=jax.ShapeDtypeStruct((_N, _ZD), jnp.float32),
    )(context, position_encoding, W, Wpe, ps)

    out = pl.pallas_call(
        _topk_kernel,
        grid=(_N // _BLOCK,),
        in_specs=[
            pl.BlockSpec((_BLOCK, _ZD), lambda i: (i, 0)),
            pl.BlockSpec((_N, _ZD), lambda i: (0, 0)),
            pl.BlockSpec((1, _BLOCK), lambda i: (0, i)),
            pl.BlockSpec((1, 1), lambda i: (0, 0)),
        ],
        out_specs=pl.BlockSpec((_BLOCK, _N), lambda i: (i, 0)),
        out_shape=jax.ShapeDtypeStruct((_N, _N), jnp.float32),
        scratch_shapes=[pltpu.VMEM((_N, _BLOCK), jnp.float32)],
        compiler_params=pltpu.CompilerParams(
            dimension_semantics=("arbitrary",),
            vmem_limit_bytes=100 * 1024 * 1024),
    )(z, z, gpr_row, os_)
    return out


# speedup vs baseline: 1.6965x; 1.0688x over previous
import jax
import jax.numpy as jnp
from jax.experimental import pallas as pl
from jax.experimental.pallas import tpu as pltpu

_N = 4096
_D = 64
_NP = 4
_NA = 32
_H = 64
_TOPK = 32
_ZD = _NP * _D + _H
_BLOCK = 512
_CHUNK = 32


def _z_kernel(ctx_ref, pe_ref, w_ref, wpe_ref, ps_ref, z_ref):
    ctx = ctx_ref[...]
    w = w_ref[...]
    for p in range(_NP):
        x = ctx * w[p, :][None, :]
        nrm = jnp.sqrt(jnp.sum(x * x, axis=1, keepdims=True))
        x = x / jnp.maximum(nrm, 1e-12)
        z_ref[:, p * _D:(p + 1) * _D] = x
    pe = jax.lax.dot_general(
        pe_ref[...], wpe_ref[...], (((1,), (0,)), ((), ())),
        preferred_element_type=jnp.float32)
    z_ref[:, _NP * _D:] = pe * ps_ref[0, 0]


def _to_key(v):
    b = jax.lax.bitcast_convert_type(v, jnp.int32)
    return b ^ jax.lax.shift_right_arithmetic(b, 31).__and__(0x7FFFFFFF)


def _to_float(k):
    b = k ^ jax.lax.shift_right_arithmetic(k, 31).__and__(0x7FFFFFFF)
    return jax.lax.bitcast_convert_type(b, jnp.float32)


def _topk_kernel(zrow_ref, zall_ref, gpr_ref, os_ref, out_ref, v_ref):
    zr = zrow_ref[...]
    za = zall_ref[...]
    dn = (((1,), (1,)), ((), ()))
    st = jax.lax.dot_general(
        za, zr, dn, preferred_element_type=jnp.float32)
    st = st * os_ref[0, 0]
    st = st * gpr_ref[...]
    v_ref[...] = st

    v3 = st.reshape(_N // _CHUNK, _CHUNK, _BLOCK)
    ninf = jnp.float32(-jnp.inf)
    c1 = jnp.max(v3, axis=1)
    e1 = v3 == c1[:, None, :]
    c2 = jnp.max(jnp.where(e1, ninf, v3), axis=1)
    e2 = jnp.logical_or(e1, v3 == c2[:, None, :])
    c3 = jnp.max(jnp.where(e2, ninf, v3), axis=1)
    cache = jnp.concatenate([c1, c2, c3], axis=0)

    ck = _to_key(cache)
    lo = jnp.min(ck, axis=0, keepdims=True)
    hi = jnp.max(ck, axis=0, keepdims=True) + 1

    def ccond(state):
        lo, hi = state
        return jnp.any(jax.lax.shift_right_logical(hi - lo, 1) != 0)

    def cbody(state):
        lo, hi = state
        half = jax.lax.shift_right_logical(hi - lo, 1)
        open_ = half != 0
        mid = lo + half
        c = jnp.sum((ck >= mid).astype(jnp.int32), axis=0, keepdims=True)
        ge = c >= _TOPK
        eq = c == _TOPK
        lo = jnp.where(jnp.logical_and(open_, ge), mid, lo)
        hi = jnp.where(
            jnp.logical_and(open_, jnp.logical_not(ge)), mid,
            jnp.where(jnp.logical_and(open_, eq), mid + 1, hi))
        return lo, hi

    lo, hi = jax.lax.while_loop(ccond, cbody, (lo, hi))
    fkey0 = lo - 1

    pinf = jnp.float32(jnp.inf)

    def rcond(state):
        fkey, tkey, open_ = state
        return jnp.any(open_ != 0)

    def rbody(state):
        fkey, tkey, open_ = state
        f = _to_float(fkey)
        v = v_ref[...]
        mask = v > f
        vm = jnp.where(mask, v, pinf)
        c = jnp.sum(mask.astype(jnp.int32), axis=0, keepdims=True)
        mn = jnp.min(vm, axis=0, keepdims=True)
        mnk = _to_key(mn)
        gt = c > _TOPK
        eq = c == _TOPK
        lt = c < _TOPK
        opn = open_ != 0
        tkey = jnp.where(jnp.logical_and(opn, eq), mnk,
                         jnp.where(jnp.logical_and(opn, lt), fkey, tkey))
        fkey = jnp.where(jnp.logical_and(opn, gt), mnk, fkey)
        open_ = jnp.where(jnp.logical_and(opn, gt), open_, 0)
        return fkey, tkey, open_

    ones = jnp.ones_like(fkey0)
    fkey, tkey, _ = jax.lax.while_loop(
        rcond, rbody, (fkey0, fkey0, ones))
    t = _to_float(tkey)
    v = v_ref[...]
    out_ref[...] = jnp.where(v >= t, v, 0.0).T


def kernel(context, position_encoding, gpr_rank, position_flag, W, Wpe):
    flag = jnp.asarray(position_flag)
    ps = jnp.where(flag == 1, 2.0, 0.0).astype(jnp.float32).reshape(1, 1)
    os_ = jnp.where(flag == 1, 0.125, 0.25).astype(jnp.float32).reshape(1, 1)
    gpr_row = gpr_rank.reshape(1, _N)

    z = pl.pallas_call(
        _z_kernel,
        out_shape=jax.ShapeDtypeStruct((_N, _ZD), jnp.float32),
    )(context, position_encoding, W, Wpe, ps)

    out = pl.pallas_call(
        _topk_kernel,
        grid=(_N // _BLOCK,),
        in_specs=[
            pl.BlockSpec((_BLOCK, _ZD), lambda i: (i, 0)),
            pl.BlockSpec((_N, _ZD), lambda i: (0, 0)),
            pl.BlockSpec((1, _BLOCK), lambda i: (0, i)),
            pl.BlockSpec((1, 1), lambda i: (0, 0)),
        ],
        out_specs=pl.BlockSpec((_BLOCK, _N), lambda i: (i, 0)),
        out_shape=jax.ShapeDtypeStruct((_N, _N), jnp.float32),
        scratch_shapes=[pltpu.VMEM((_N, _BLOCK), jnp.float32)],
        compiler_params=pltpu.CompilerParams(
            dimension_semantics=("arbitrary",),
            vmem_limit_bytes=100 * 1024 * 1024),
    )(z, z, gpr_row, os_)
    return out
